# baseline (speedup 1.0000x reference)
import jax
import jax.numpy as jnp
from jax import lax
from jax.experimental import pallas as pl
from jax.experimental.pallas import tpu as pltpu
from jax.experimental.pallas import tpu_sc as plsc

N_NODES = 10000
N_EDGES = 320000
D_IN, D_HID, D_LAT = 128, 64, 32
N_GRAPHS = 64

NB = 10240
NC, NS = 2, 16
NW = NC * NS
CHUNK = 125
KCH = 80
RPT = NB // NS
DUMMY = N_NODES
ROW_BLK = 2048


def _mesh():
    return plsc.VectorSubcoreMesh(
        core_axis_name="c", subcore_axis_name="s", num_cores=NC, num_subcores=NS
    )


def _deg_body(dst_hbm, counts_hbm, dst_v, ones_v, zero_v, acc_sh):
    c = lax.axis_index("c")
    s = lax.axis_index("s")

    def fill_ones(i, _):
        ones_v[pl.ds(i * 16, 16)] = jnp.full((16,), 1.0, jnp.float32)
        return 0

    lax.fori_loop(0, CHUNK // 16, fill_ones, 0)

    def fill_zero(i, _):
        zero_v[pl.ds(i * 16, 16)] = jnp.zeros((16,), jnp.float32)
        return 0

    lax.fori_loop(0, RPT // 16, fill_zero, 0)

    pltpu.sync_copy(zero_v, acc_sh.at[pl.ds(s * RPT, RPT)])
    pltpu.sync_copy(dst_hbm.at[c, s], dst_v)
    plsc.subcore_barrier()

    def body(j, _):
        pltpu.sync_copy(ones_v, acc_sh.at[dst_v.at[j]], add=True)
        return 0

    lax.fori_loop(0, KCH, body, 0)
    plsc.subcore_barrier()
    pltpu.sync_copy(
        acc_sh.at[pl.ds(s * RPT, RPT)], counts_hbm.at[c, pl.ds(s * RPT, RPT)]
    )


def _deg(dst_p):
    f = pl.kernel(
        _deg_body,
        out_type=jax.ShapeDtypeStruct((NC, NB), jnp.float32),
        mesh=_mesh(),
        scratch_types=[
            pltpu.VMEM((KCH, CHUNK), jnp.int32),
            pltpu.VMEM((CHUNK,), jnp.float32),
            pltpu.VMEM((RPT,), jnp.float32),
            pltpu.VMEM_SHARED((NB,), jnp.float32),
        ],
    )
    return f(dst_p)


def _agg_body(hs_hbm, src_hbm, dst_hbm, part_hbm,
              src_v, dst_v, msg, zero_v, acc_sh, gsem, ssem):
    d = zero_v.shape[1]
    nbuf = msg.shape[0]
    ngrp = KCH // nbuf
    c = lax.axis_index("c")
    s = lax.axis_index("s")

    def fz(i, _):
        def fz2(k, _):
            zero_v[i, pl.ds(k * 16, 16)] = jnp.zeros((16,), jnp.float32)
            return 0
        lax.fori_loop(0, d // 16, fz2, 0)
        return 0

    lax.fori_loop(0, CHUNK, fz, 0)
    for r in range(RPT // CHUNK):
        pltpu.sync_copy(zero_v, acc_sh.at[pl.ds(s * RPT + r * CHUNK, CHUNK)])

    pltpu.sync_copy(src_hbm.at[c, s], src_v)
    pltpu.sync_copy(dst_hbm.at[c, s], dst_v)
    plsc.subcore_barrier()

    for b in range(nbuf):
        pltpu.async_copy(hs_hbm.at[src_v.at[b]], msg.at[b], gsem.at[b])

    def body(g, _):
        for b in range(nbuf):
            j = g * nbuf + b
            pltpu.make_async_copy(
                hs_hbm.at[src_v.at[j]], msg.at[b], gsem.at[b]
            ).wait()
            pltpu.async_copy(
                msg.at[b], acc_sh.at[dst_v.at[j]], ssem.at[b], add=True
            )
        for b in range(nbuf):
            j = g * nbuf + b
            pltpu.make_async_copy(
                msg.at[b], acc_sh.at[dst_v.at[j]], ssem.at[b]
            ).wait()

            @pl.when(g < ngrp - 1)
            def _():
                pltpu.async_copy(
                    hs_hbm.at[src_v.at[j + nbuf]], msg.at[b], gsem.at[b]
                )
        return 0

    lax.fori_loop(0, ngrp, body, 0)
    plsc.subcore_barrier()
    pltpu.sync_copy(
        acc_sh.at[pl.ds(s * RPT, RPT)], part_hbm.at[c, pl.ds(s * RPT, RPT)]
    )


def _agg(hs, src_p, dst_p, d, nbuf):
    f = pl.kernel(
        _agg_body,
        out_type=jax.ShapeDtypeStruct((NC, NB, d), jnp.float32),
        mesh=_mesh(),
        compiler_params=pltpu.CompilerParams(use_tc_tiling_on_sc=False),
        scratch_types=[
            pltpu.VMEM((KCH, CHUNK), jnp.int32),
            pltpu.VMEM((KCH, CHUNK), jnp.int32),
            pltpu.VMEM((nbuf, CHUNK, d), jnp.float32),
            pltpu.VMEM((CHUNK, d), jnp.float32),
            pltpu.VMEM_SHARED((NB, d), jnp.float32),
            pltpu.SemaphoreType.DMA((nbuf,)),
            pltpu.SemaphoreType.DMA((nbuf,)),
        ],
    )
    return f(hs, src_p, dst_p)


def _mm1_body(x_ref, w_ref, cnt_ref, hs_ref, dis_ref):
    i = pl.program_id(0)
    dis = lax.rsqrt(cnt_ref[0] + cnt_ref[1] + 1.0)
    h = jnp.dot(x_ref[...], w_ref[...], preferred_element_type=jnp.float32)
    row = i * ROW_BLK + lax.broadcasted_iota(jnp.int32, (ROW_BLK, 1), 0)
    hs_ref[...] = jnp.where(row < N_NODES, h * dis, 0.0)
    dis_ref[...] = dis


def _mm1(x, W1, counts3):
    return pl.pallas_call(
        _mm1_body,
        grid=(NB // ROW_BLK,),
        in_specs=[
            pl.BlockSpec((ROW_BLK, D_IN), lambda i: (i, 0)),
            pl.BlockSpec((D_IN, D_HID), lambda i: (0, 0)),
            pl.BlockSpec((NC, ROW_BLK, 1), lambda i: (0, i, 0)),
        ],
        out_specs=[
            pl.BlockSpec((ROW_BLK, D_HID), lambda i: (i, 0)),
            pl.BlockSpec((ROW_BLK, 1), lambda i: (i, 0)),
        ],
        out_shape=[
            jax.ShapeDtypeStruct((NB, D_HID), jnp.float32),
            jax.ShapeDtypeStruct((NB, 1), jnp.float32),
        ],
    )(x, W1, counts3)


def _mid_body(p_ref, hs1_ref, dis_ref, b1_ref, w2_ref, hs2_ref):
    dis = dis_ref[...]
    t = (p_ref[0] + p_ref[1] + hs1_ref[...]) * dis + b1_ref[...]
    h = jnp.maximum(t, 0.0)
    hs2_ref[...] = (
        jnp.dot(h, w2_ref[...], preferred_element_type=jnp.float32) * dis
    )


def _mid(p1, hs1, dis, b1r, W2):
    return pl.pallas_call(
        _mid_body,
        grid=(NB // ROW_BLK,),
        in_specs=[
            pl.BlockSpec((NC, ROW_BLK, D_HID), lambda i: (0, i, 0)),
            pl.BlockSpec((ROW_BLK, D_HID), lambda i: (i, 0)),
            pl.BlockSpec((ROW_BLK, 1), lambda i: (i, 0)),
            pl.BlockSpec((1, D_HID), lambda i: (0, 0)),
            pl.BlockSpec((D_HID, D_LAT), lambda i: (0, 0)),
        ],
        out_specs=pl.BlockSpec((ROW_BLK, D_LAT), lambda i: (i, 0)),
        out_shape=jax.ShapeDtypeStruct((NB, D_LAT), jnp.float32),
    )(p1, hs1, dis, b1r, W2)


def _pool_body(q_ref, hs2_ref, dis_ref, b2_ref, bat_ref, out_ref, acc_ref):
    i = pl.program_id(0)

    @pl.when(i == 0)
    def _():
        acc_ref[...] = jnp.zeros_like(acc_ref)

    z = (q_ref[0] + q_ref[1] + hs2_ref[...]) * dis_ref[...] + b2_ref[...]
    z1 = jnp.concatenate(
        [z, jnp.ones((ROW_BLK, 1), jnp.float32)], axis=1
    )
    m = (bat_ref[...] == lax.broadcasted_iota(jnp.int32, (1, N_GRAPHS), 1))
    m = m.astype(jnp.float32)
    acc_ref[...] += lax.dot_general(
        m, z1, (((0,), (0,)), ((), ())), preferred_element_type=jnp.float32
    )

    @pl.when(i == pl.num_programs(0) - 1)
    def _():
        a = acc_ref[...]
        out_ref[...] = a[:, :D_LAT] / jnp.maximum(a[:, D_LAT:D_LAT + 1], 1.0)


def _pool(p2, hs2, dis, b2r, bat):
    return pl.pallas_call(
        _pool_body,
        grid=(NB // ROW_BLK,),
        in_specs=[
            pl.BlockSpec((NC, ROW_BLK, D_LAT), lambda i: (0, i, 0)),
            pl.BlockSpec((ROW_BLK, D_LAT), lambda i: (i, 0)),
            pl.BlockSpec((ROW_BLK, 1), lambda i: (i, 0)),
            pl.BlockSpec((1, D_LAT), lambda i: (0, 0)),
            pl.BlockSpec((ROW_BLK, 1), lambda i: (i, 0)),
        ],
        out_specs=pl.BlockSpec((N_GRAPHS, D_LAT), lambda i: (0, 0)),
        out_shape=jax.ShapeDtypeStruct((N_GRAPHS, D_LAT), jnp.float32),
        scratch_shapes=[pltpu.VMEM((N_GRAPHS, D_LAT + 1), jnp.float32)],
    )(p2, hs2, dis, b2r, bat)


@jax.jit
def kernel(x, edge_index, batch, W1, b1, W2, b2):
    src_p = edge_index[0].astype(jnp.int32).reshape(NC, NS, KCH, CHUNK)
    dst_p = edge_index[1].astype(jnp.int32).reshape(NC, NS, KCH, CHUNK)
    bat = jnp.pad(
        batch.astype(jnp.int32), (0, NB - N_NODES), constant_values=N_GRAPHS
    ).reshape(NB, 1)

    counts = _deg(dst_p)
    hs1, dis = _mm1(x, W1, counts.reshape(NC, NB, 1))
    p1 = _agg(hs1, src_p, dst_p, D_HID, 5)
    hs2 = _mid(p1, hs1, dis, b1.reshape(1, D_HID), W2)
    p2 = _agg(hs2, src_p, dst_p, D_LAT, 8)
    return _pool(p2, hs2, dis, b2.reshape(1, D_LAT), bat)


# speedup vs baseline: 1.0085x; 1.0085x over previous
import jax
import jax.numpy as jnp
from jax import lax
from jax.experimental import pallas as pl
from jax.experimental.pallas import tpu as pltpu
from jax.experimental.pallas import tpu_sc as plsc

N_NODES = 10000
N_EDGES = 320000
D_IN, D_HID, D_LAT = 128, 64, 32
N_GRAPHS = 64

NB = 10240
NC, NS = 2, 16
NW = NC * NS
CHUNK = 125
KCH = 80
RPT = NB // NS
DUMMY = N_NODES
ROW_BLK = 5120


def _mesh():
    return plsc.VectorSubcoreMesh(
        core_axis_name="c", subcore_axis_name="s", num_cores=NC, num_subcores=NS
    )


def _deg_body(dst_hbm, counts_hbm, dst_v, ones_v, zero_v, acc_sh):
    c = lax.axis_index("c")
    s = lax.axis_index("s")

    def fill_ones(i, _):
        ones_v[pl.ds(i * 16, 16)] = jnp.full((16,), 1.0, jnp.float32)
        return 0

    lax.fori_loop(0, CHUNK // 16, fill_ones, 0)

    def fill_zero(i, _):
        zero_v[pl.ds(i * 16, 16)] = jnp.zeros((16,), jnp.float32)
        return 0

    lax.fori_loop(0, RPT // 16, fill_zero, 0)

    pltpu.sync_copy(zero_v, acc_sh.at[pl.ds(s * RPT, RPT)])
    pltpu.sync_copy(dst_hbm.at[c, s], dst_v)
    plsc.subcore_barrier()

    def body(j, _):
        pltpu.sync_copy(ones_v, acc_sh.at[dst_v.at[j]], add=True)
        return 0

    lax.fori_loop(0, KCH, body, 0)
    plsc.subcore_barrier()
    pltpu.sync_copy(
        acc_sh.at[pl.ds(s * RPT, RPT)], counts_hbm.at[c, pl.ds(s * RPT, RPT)]
    )


def _deg(dst_p):
    f = pl.kernel(
        _deg_body,
        out_type=jax.ShapeDtypeStruct((NC, NB), jnp.float32),
        mesh=_mesh(),
        scratch_types=[
            pltpu.VMEM((KCH, CHUNK), jnp.int32),
            pltpu.VMEM((CHUNK,), jnp.float32),
            pltpu.VMEM((RPT,), jnp.float32),
            pltpu.VMEM_SHARED((NB,), jnp.float32),
        ],
    )
    return f(dst_p)


def _agg_body(hs_hbm, src_hbm, dst_hbm, part_hbm,
              src_v, dst_v, msg, zero_v, acc_sh, gsem, ssem):
    d = zero_v.shape[1]
    nbuf = msg.shape[0]
    ngrp = KCH // nbuf
    c = lax.axis_index("c")
    s = lax.axis_index("s")

    def fz(i, _):
        def fz2(k, _):
            zero_v[i, pl.ds(k * 16, 16)] = jnp.zeros((16,), jnp.float32)
            return 0
        lax.fori_loop(0, d // 16, fz2, 0)
        return 0

    lax.fori_loop(0, CHUNK, fz, 0)
    for r in range(RPT // CHUNK):
        pltpu.sync_copy(zero_v, acc_sh.at[pl.ds(s * RPT + r * CHUNK, CHUNK)])

    pltpu.sync_copy(src_hbm.at[c, s], src_v)
    pltpu.sync_copy(dst_hbm.at[c, s], dst_v)
    plsc.subcore_barrier()

    for b in range(nbuf):
        pltpu.async_copy(hs_hbm.at[src_v.at[b]], msg.at[b], gsem.at[b])

    def body(g, _):
        for b in range(nbuf):
            j = g * nbuf + b
            pltpu.make_async_copy(
                hs_hbm.at[src_v.at[j]], msg.at[b], gsem.at[b]
            ).wait()
            pltpu.async_copy(
                msg.at[b], acc_sh.at[dst_v.at[j]], ssem.at[b], add=True
            )
        for b in range(nbuf):
            j = g * nbuf + b
            pltpu.make_async_copy(
                msg.at[b], acc_sh.at[dst_v.at[j]], ssem.at[b]
            ).wait()

            @pl.when(g < ngrp - 1)
            def _():
                pltpu.async_copy(
                    hs_hbm.at[src_v.at[j + nbuf]], msg.at[b], gsem.at[b]
                )
        return 0

    lax.fori_loop(0, ngrp, body, 0)
    plsc.subcore_barrier()
    pltpu.sync_copy(
        acc_sh.at[pl.ds(s * RPT, RPT)], part_hbm.at[c, pl.ds(s * RPT, RPT)]
    )


def _agg(hs, src_p, dst_p, d, nbuf):
    f = pl.kernel(
        _agg_body,
        out_type=jax.ShapeDtypeStruct((NC, NB, d), jnp.float32),
        mesh=_mesh(),
        compiler_params=pltpu.CompilerParams(use_tc_tiling_on_sc=False),
        scratch_types=[
            pltpu.VMEM((KCH, CHUNK), jnp.int32),
            pltpu.VMEM((KCH, CHUNK), jnp.int32),
            pltpu.VMEM((nbuf, CHUNK, d), jnp.float32),
            pltpu.VMEM((CHUNK, d), jnp.float32),
            pltpu.VMEM_SHARED((NB, d), jnp.float32),
            pltpu.SemaphoreType.DMA((nbuf,)),
            pltpu.SemaphoreType.DMA((nbuf,)),
        ],
    )
    return f(hs, src_p, dst_p)


def _mm1_body(x_ref, w_ref, cnt_ref, hs_ref, dis_ref):
    i = pl.program_id(0)
    dis = lax.rsqrt(cnt_ref[0] + cnt_ref[1] + 1.0)
    h = jnp.dot(x_ref[...], w_ref[...], preferred_element_type=jnp.float32)
    row = i * ROW_BLK + lax.broadcasted_iota(jnp.int32, (ROW_BLK, 1), 0)
    hs_ref[...] = jnp.where(row < N_NODES, h * dis, 0.0)
    dis_ref[...] = dis


def _mm1(x, W1, counts3):
    return pl.pallas_call(
        _mm1_body,
        grid=(NB // ROW_BLK,),
        in_specs=[
            pl.BlockSpec((ROW_BLK, D_IN), lambda i: (i, 0)),
            pl.BlockSpec((D_IN, D_HID), lambda i: (0, 0)),
            pl.BlockSpec((NC, ROW_BLK, 1), lambda i: (0, i, 0)),
        ],
        out_specs=[
            pl.BlockSpec((ROW_BLK, D_HID), lambda i: (i, 0)),
            pl.BlockSpec((ROW_BLK, 1), lambda i: (i, 0)),
        ],
        out_shape=[
            jax.ShapeDtypeStruct((NB, D_HID), jnp.float32),
            jax.ShapeDtypeStruct((NB, 1), jnp.float32),
        ],
    )(x, W1, counts3)


def _mid_body(p_ref, hs1_ref, dis_ref, b1_ref, w2_ref, hs2_ref):
    dis = dis_ref[...]
    t = (p_ref[0] + p_ref[1] + hs1_ref[...]) * dis + b1_ref[...]
    h = jnp.maximum(t, 0.0)
    hs2_ref[...] = (
        jnp.dot(h, w2_ref[...], preferred_element_type=jnp.float32) * dis
    )


def _mid(p1, hs1, dis, b1r, W2):
    return pl.pallas_call(
        _mid_body,
        grid=(NB // ROW_BLK,),
        in_specs=[
            pl.BlockSpec((NC, ROW_BLK, D_HID), lambda i: (0, i, 0)),
            pl.BlockSpec((ROW_BLK, D_HID), lambda i: (i, 0)),
            pl.BlockSpec((ROW_BLK, 1), lambda i: (i, 0)),
            pl.BlockSpec((1, D_HID), lambda i: (0, 0)),
            pl.BlockSpec((D_HID, D_LAT), lambda i: (0, 0)),
        ],
        out_specs=pl.BlockSpec((ROW_BLK, D_LAT), lambda i: (i, 0)),
        out_shape=jax.ShapeDtypeStruct((NB, D_LAT), jnp.float32),
    )(p1, hs1, dis, b1r, W2)


def _pool_body(q_ref, hs2_ref, dis_ref, b2_ref, bat_ref, out_ref, acc_ref):
    i = pl.program_id(0)

    @pl.when(i == 0)
    def _():
        acc_ref[...] = jnp.zeros_like(acc_ref)

    z = (q_ref[0] + q_ref[1] + hs2_ref[...]) * dis_ref[...] + b2_ref[...]
    z1 = jnp.concatenate(
        [z, jnp.ones((ROW_BLK, 1), jnp.float32)], axis=1
    )
    m = (bat_ref[...] == lax.broadcasted_iota(jnp.int32, (1, N_GRAPHS), 1))
    m = m.astype(jnp.float32)
    acc_ref[...] += lax.dot_general(
        m, z1, (((0,), (0,)), ((), ())), preferred_element_type=jnp.float32
    )

    @pl.when(i == pl.num_programs(0) - 1)
    def _():
        a = acc_ref[...]
        out_ref[...] = a[:, :D_LAT] / jnp.maximum(a[:, D_LAT:D_LAT + 1], 1.0)


def _pool(p2, hs2, dis, b2r, bat):
    return pl.pallas_call(
        _pool_body,
        grid=(NB // ROW_BLK,),
        in_specs=[
            pl.BlockSpec((NC, ROW_BLK, D_LAT), lambda i: (0, i, 0)),
            pl.BlockSpec((ROW_BLK, D_LAT), lambda i: (i, 0)),
            pl.BlockSpec((ROW_BLK, 1), lambda i: (i, 0)),
            pl.BlockSpec((1, D_LAT), lambda i: (0, 0)),
            pl.BlockSpec((ROW_BLK, 1), lambda i: (i, 0)),
        ],
        out_specs=pl.BlockSpec((N_GRAPHS, D_LAT), lambda i: (0, 0)),
        out_shape=jax.ShapeDtypeStruct((N_GRAPHS, D_LAT), jnp.float32),
        scratch_shapes=[pltpu.VMEM((N_GRAPHS, D_LAT + 1), jnp.float32)],
    )(p2, hs2, dis, b2r, bat)


@jax.jit
def kernel(x, edge_index, batch, W1, b1, W2, b2):
    src_p = edge_index[0].astype(jnp.int32).reshape(NC, NS, KCH, CHUNK)
    dst_p = edge_index[1].astype(jnp.int32).reshape(NC, NS, KCH, CHUNK)
    bat = jnp.pad(
        batch.astype(jnp.int32), (0, NB - N_NODES), constant_values=N_GRAPHS
    ).reshape(NB, 1)

    counts = _deg(dst_p)
    hs1, dis = _mm1(x, W1, counts.reshape(NC, NB, 1))
    p1 = _agg(hs1, src_p, dst_p, D_HID, 5)
    hs2 = _mid(p1, hs1, dis, b1.reshape(1, D_HID), W2)
    p2 = _agg(hs2, src_p, dst_p, D_LAT, 8)
    return _pool(p2, hs2, dis, b2.reshape(1, D_LAT), bat)
